# baseline (speedup 1.0000x reference)
import jax
import jax.numpy as jnp
from jax.experimental import pallas as pl
from jax.experimental.pallas import tpu as pltpu

_BJ = 512
_NI = 2


def _fused_body(adj_ref, x_ref, u_ref, out_ref, deg_ref):
    j = pl.program_id(1)
    nj = pl.num_programs(1)

    a = adj_ref[...]
    m = (a > 0).astype(jnp.bfloat16)
    xb = x_ref[pl.ds(j * _BJ, _BJ), :]
    acc = jax.lax.dot_general(
        m, xb, (((0,), (0,)), ((), ())),
        preferred_element_type=jnp.float32)
    dsum = jnp.sum(a, axis=0, keepdims=True)

    @pl.when(j == 0)
    def _first():
        out_ref[...] = acc
        deg_ref[...] = dsum

    @pl.when(j > 0)
    def _rest():
        out_ref[...] += acc
        deg_ref[...] += dsum

    @pl.when(j == nj - 1)
    def _epilogue():
        r = jnp.transpose(1.0 / deg_ref[...], (1, 0))
        agg = (out_ref[...] * r).astype(jnp.bfloat16)
        h = jax.lax.dot_general(
            agg, u_ref[...], (((1,), (1,)), ((), ())),
            preferred_element_type=jnp.float32)
        out_ref[...] = jnp.maximum(h, 0.0)


def kernel(x, adj_mat, U):
    n, d = x.shape
    bi = n // _NI
    xb16 = x.astype(jnp.bfloat16)
    ub16 = U.astype(jnp.bfloat16)
    out = pl.pallas_call(
        _fused_body,
        grid=(_NI, n // _BJ),
        in_specs=[
            pl.BlockSpec((_BJ, bi), lambda i, j: (j, i)),
            pl.BlockSpec((n, d), lambda i, j: (0, 0)),
            pl.BlockSpec((d, d), lambda i, j: (0, 0)),
        ],
        out_specs=pl.BlockSpec((bi, d), lambda i, j: (i, 0)),
        out_shape=jax.ShapeDtypeStruct((n, d), jnp.float32),
        scratch_shapes=[pltpu.VMEM((1, bi), jnp.float32)],
        compiler_params=pltpu.CompilerParams(
            dimension_semantics=("parallel", "arbitrary")),
    )(adj_mat, xb16, ub16)
    return out[None, :, :]


# speedup vs baseline: 1.2009x; 1.2009x over previous
import jax
import jax.numpy as jnp
from jax.experimental import pallas as pl
from jax.experimental.pallas import tpu as pltpu

_BJ = 512


def _fused_body(adj_ref, x_ref, u_ref, out_ref, agg_ref, deg_ref):
    j = pl.program_id(0)
    nj = pl.num_programs(0)

    a = adj_ref[...]
    m = (a > 0).astype(jnp.bfloat16)
    xb = x_ref[pl.ds(j * _BJ, _BJ), :]
    acc = jax.lax.dot_general(
        xb, m, (((0,), (0,)), ((), ())),
        preferred_element_type=jnp.float32)
    dsum = jnp.sum(a, axis=0, keepdims=True)

    @pl.when(j == 0)
    def _first():
        agg_ref[...] = acc
        deg_ref[...] = dsum

    @pl.when(j > 0)
    def _rest():
        agg_ref[...] += acc
        deg_ref[...] += dsum

    @pl.when(j == nj - 1)
    def _epilogue():
        r = 1.0 / deg_ref[...]
        aggs = (agg_ref[...] * r).astype(jnp.bfloat16)
        h = jax.lax.dot_general(
            aggs, u_ref[...], (((0,), (1,)), ((), ())),
            preferred_element_type=jnp.float32)
        out_ref[...] = jnp.maximum(h, 0.0)


def kernel(x, adj_mat, U):
    n, d = x.shape
    xb16 = x.astype(jnp.bfloat16)
    ub16 = U.astype(jnp.bfloat16)
    out = pl.pallas_call(
        _fused_body,
        grid=(n // _BJ,),
        in_specs=[
            pl.BlockSpec((_BJ, n), lambda j: (j, 0)),
            pl.BlockSpec((n, d), lambda j: (0, 0)),
            pl.BlockSpec((d, d), lambda j: (0, 0)),
        ],
        out_specs=pl.BlockSpec((n, d), lambda j: (0, 0)),
        out_shape=jax.ShapeDtypeStruct((n, d), jnp.float32),
        scratch_shapes=[pltpu.VMEM((d, n), jnp.float32),
                        pltpu.VMEM((1, n), jnp.float32)],
        compiler_params=pltpu.CompilerParams(
            dimension_semantics=("arbitrary",)),
    )(adj_mat, xb16, ub16)
    return out[None, :, :]
